# baseline (speedup 1.0000x reference)
import functools

import jax
import jax.numpy as jnp
from jax import lax
from jax.experimental import pallas as pl
from jax.experimental.pallas import tpu as pltpu
from jax.experimental.pallas import tpu_sc as plsc

EMB_DIM = 16
VOCAB = 1000000
SUP_ROWS = VOCAB * EMB_DIM // 128
BATCH = 16384
SLOTS = 26
NUM_WORKERS = 32
NBLK = BATCH // 128
BPW = NBLK // NUM_WORKERS
TILES_PW = SLOTS * BPW // 2
MAGIC = 12582912.0
QLOW = -128.0
QHIGH = 127.0

_mesh = plsc.VectorSubcoreMesh(core_axis_name="c", subcore_axis_name="s")

TCOL = 896
NTILE_T = VOCAB // TCOL
TAIL_COLS = VOCAB - NTILE_T * TCOL
TITER = (NTILE_T + NUM_WORKERS - 1) // NUM_WORKERS


@functools.partial(
    pl.kernel,
    out_type=jax.ShapeDtypeStruct((SUP_ROWS, 128), jnp.float32),
    mesh=_mesh,
    scratch_types=[
        pltpu.VMEM((2, EMB_DIM, TCOL), jnp.float32),
        pltpu.VMEM((2, TCOL // 8, 128), jnp.float32),
        pltpu.VMEM((EMB_DIM, TAIL_COLS), jnp.float32),
        pltpu.SemaphoreType.DMA,
        pltpu.SemaphoreType.DMA,
        pltpu.SemaphoreType.DMA,
        pltpu.SemaphoreType.DMA,
    ],
    compiler_params=pltpu.CompilerParams(needs_layout_passes=False),
)
def _transpose_table(wt_hbm, w2_hbm, tin_v, tout_v, tail_v, l0, l1, s0, s1):
    wid = lax.axis_index("s") * 2 + lax.axis_index("c")
    iota = lax.iota(jnp.int32, 16)
    lsems = (l0, l1)
    ssems = (s0, s1)
    f16 = [(iota + j) & 15 for j in range(16)]
    rowo = [((iota + j) & 15) >> 3 for j in range(16)]
    colw = [(((iota + j) & 15) & 7) * 16 + iota for j in range(16)]

    def c0_of(k):
        return pl.multiple_of((wid + NUM_WORKERS * k) * TCOL, TCOL)

    def sp0_of(k):
        return pl.multiple_of((wid + NUM_WORKERS * k) * (TCOL // 8),
                              TCOL // 8)

    def valid(k):
        return (wid + NUM_WORKERS * k) < NTILE_T

    def start_load(k, slot):
        pltpu.async_copy(wt_hbm.at[:, pl.ds(c0_of(k), TCOL)],
                         tin_v.at[slot], lsems[slot])

    def wait_load(k, slot):
        pltpu.make_async_copy(wt_hbm.at[:, pl.ds(c0_of(k), TCOL)],
                              tin_v.at[slot], lsems[slot]).wait()

    def start_store(k, slot):
        pltpu.async_copy(tout_v.at[slot],
                         w2_hbm.at[pl.ds(sp0_of(k), TCOL // 8)],
                         ssems[slot])

    def wait_store(k, slot):
        pltpu.make_async_copy(tout_v.at[slot],
                              w2_hbm.at[pl.ds(sp0_of(k), TCOL // 8)],
                              ssems[slot]).wait()

    @pl.when(valid(0))
    def _():
        start_load(0, 0)

    @pl.when(valid(1))
    def _():
        start_load(1, 1)

    def pair(p, _):
        for slot in (0, 1):
            k = 2 * p + slot

            @pl.when(valid(k))
            def _():
                wait_load(k, slot)

                @pl.when(k >= 2)
                def _():
                    wait_store(k - 2, slot)

                def shuffle(sp2, _):
                    for j in range(16):
                        v = plsc.load_gather(tin_v.at[slot],
                                             [iota, sp2 * 16 + f16[j]])
                        plsc.store_scatter(tout_v.at[slot],
                                           [sp2 * 2 + rowo[j], colw[j]], v)
                    return 0
                lax.fori_loop(0, TCOL // 16, shuffle, 0, unroll=2)

                @pl.when(valid(k + 2))
                def _():
                    start_load(k + 2, slot)

                start_store(k, slot)
        return 0

    lax.fori_loop(0, (TITER + 1) // 2, pair, 0)

    @pl.when(valid(TITER - 2))
    def _():
        wait_store(TITER - 2, (TITER - 2) % 2)

    @pl.when(valid(TITER - 1))
    def _():
        wait_store(TITER - 1, (TITER - 1) % 2)

    @pl.when(wid == 0)
    def _():
        pltpu.sync_copy(wt_hbm.at[:, pl.ds(NTILE_T * TCOL, TAIL_COLS)],
                        tail_v)
        for sp2 in range(TAIL_COLS // 16):
            for j in range(16):
                v = plsc.load_gather(tail_v, [iota, sp2 * 16 + f16[j]])
                plsc.store_scatter(tout_v.at[0],
                                   [sp2 * 2 + rowo[j], colw[j]], v)
        pltpu.sync_copy(tout_v.at[0, pl.ds(0, TAIL_COLS // 8)],
                        w2_hbm.at[pl.ds(NTILE_T * (TCOL // 8),
                                        TAIL_COLS // 8)])


@functools.partial(
    pl.kernel,
    out_type=jax.ShapeDtypeStruct((SLOTS, 2, NBLK, 8, 128), jnp.float32),
    mesh=_mesh,
    scratch_types=[
        pltpu.VMEM((SLOTS, BPW, 128), jnp.int32),
        pltpu.VMEM((2, 2, 128), jnp.int32),
        pltpu.VMEM((2, 256), jnp.int32),
        pltpu.VMEM((2, 256, 128), jnp.float32),
        pltpu.VMEM((2, EMB_DIM, 256), jnp.float32),
        pltpu.VMEM((32,), jnp.float32),
        pltpu.SemaphoreType.DMA,
        pltpu.SemaphoreType.DMA,
        pltpu.SemaphoreType.DMA,
        pltpu.SemaphoreType.DMA,
        pltpu.SemaphoreType.DMA,
    ],
    compiler_params=pltpu.CompilerParams(needs_layout_passes=False),
)
def _lsq_lookup(xt_hbm, w_hbm, scale_hbm, out_hbm,
                idx_v, sup_v, sub_v, super_v, res_v, scale_v,
                isem, gsem0, gsem1, osem0, osem1):
    wid = lax.axis_index("s") * 2 + lax.axis_index("c")
    col0 = wid * (BPW * 128)

    pltpu.sync_copy(scale_hbm, scale_v)
    for j in range(BPW):
        pltpu.async_copy(
            xt_hbm.at[:, pl.ds(col0 + j * 128, 128)], idx_v.at[:, j, :],
            isem)
    for j in range(BPW):
        pltpu.make_async_copy(
            xt_hbm.at[:, pl.ds(col0 + j * 128, 128)], idx_v.at[:, j, :],
            isem).wait()
    inv_a = scale_v[pl.ds(0, 16)]
    a = scale_v[pl.ds(16, 16)]

    gsems = (gsem0, gsem1)
    osems = (osem0, osem1)

    def supsub(t, slot):
        s = lax.div(t, 2)
        jp = lax.rem(t, 2)
        for jj in range(2):
            def body(i, _):
                v = idx_v[s, jp * 2 + jj, pl.ds(i * 16, 16)]
                sup_v[slot, jj, pl.ds(i * 16, 16)] = (
                    lax.shift_right_logical(v, 3))
                sub_v[slot, pl.ds(jj * 128 + i * 16, 16)] = (v & 7) * 16
                return 0
            lax.fori_loop(0, 8, body, 0, unroll=4)

    def start_gather(slot):
        for jj in range(2):
            pltpu.async_copy(
                w_hbm.at[sup_v.at[slot, jj]],
                super_v.at[slot, pl.ds(jj * 128, 128)], gsems[slot])

    def wait_gather(slot):
        for jj in range(2):
            pltpu.make_async_copy(
                w_hbm.at[sup_v.at[slot, jj]],
                super_v.at[slot, pl.ds(jj * 128, 128)], gsems[slot]).wait()

    def start_out(t, slot):
        s = lax.div(t, 2)
        jp = lax.rem(t, 2)
        for eb in range(2):
            for jj in range(2):
                pltpu.async_copy(
                    res_v.at[slot, pl.ds(eb * 8, 8), pl.ds(jj * 128, 128)],
                    out_hbm.at[s, eb, wid * BPW + jp * 2 + jj],
                    osems[slot])

    def wait_out(t, slot):
        s = lax.div(t, 2)
        jp = lax.rem(t, 2)
        for eb in range(2):
            for jj in range(2):
                pltpu.make_async_copy(
                    res_v.at[slot, pl.ds(eb * 8, 8), pl.ds(jj * 128, 128)],
                    out_hbm.at[s, eb, wid * BPW + jp * 2 + jj],
                    osems[slot]).wait()

    iota = lax.iota(jnp.int32, 16)
    feat = [(iota + k) & 15 for k in range(EMB_DIM)]

    def extract_quant(slot):
        def body(i16, _):
            local = i16 * 16 + iota
            s16 = sub_v[slot, pl.ds(i16 * 16, 16)]
            for k in range(EMB_DIM):
                v = plsc.load_gather(super_v.at[slot],
                                     [local, s16 + feat[k]])
                r = (v * inv_a + MAGIC) - MAGIC
                r = jnp.minimum(jnp.maximum(r, QLOW), QHIGH)
                plsc.store_scatter(res_v.at[slot], [feat[k], local], r * a)
            return 0
        lax.fori_loop(0, 16, body, 0)

    supsub(0, 0)
    start_gather(0)
    supsub(1, 1)
    start_gather(1)

    def pair(p, _):
        for slot in (0, 1):
            t = 2 * p + slot
            wait_gather(slot)

            @pl.when(p >= 1)
            def _():
                wait_out(t - 2, slot)

            extract_quant(slot)

            @pl.when(p < TILES_PW // 2 - 1)
            def _():
                supsub(t + 2, slot)
                start_gather(slot)

            start_out(t, slot)
        return 0

    lax.fori_loop(0, TILES_PW // 2, pair, 0)
    wait_out(TILES_PW - 2, 0)
    wait_out(TILES_PW - 1, 1)


def kernel(x, weight, alpha):
    a = jnp.abs(alpha).astype(jnp.float32) + 1e-10
    scale = jnp.concatenate([jnp.full((16,), 1.0 / a, jnp.float32),
                             jnp.full((16,), a, jnp.float32)])
    xt = x.T.astype(jnp.int32)
    w2 = _transpose_table(weight.T)
    out5 = _lsq_lookup(xt, w2, scale)
    return out5.transpose(2, 4, 0, 1, 3).reshape(BATCH, SLOTS, EMB_DIM)
